# baseline (speedup 1.0000x reference)
import functools

import jax
import jax.numpy as jnp
from jax import lax
from jax.experimental import pallas as pl
from jax.experimental.pallas import tpu as pltpu
from jax.experimental.pallas import tpu_sc as plsc

_S = 50
_C = 20
_D = 64
_B = 1024
_NW = 32
_NV = _B * _S
_RPW = _B // _NW
_SUB = 128
_S0 = (0, 16, 32)
_SZ = (16, 16, 18)
_NK = 4
_IDXPAD = 1280


def _splits(n_rows):
    out, off = [], 0
    while off < n_rows:
        n = min(_SUB, n_rows - off)
        out.append((off, n))
        off += n
    return out


def _sc_body(codes_hbm, lens_hbm, table_hbm, out_hbm,
             idx_v, ctw_v, a_v, b_v, c_v, out_v, lens_v, recip_v, n0_v, t0_v,
             gsema, gsemb, gsemc, csem, osem0, osem1):
    nc = 2
    wid = lax.axis_index("s") * nc + lax.axis_index("c")
    b0 = wid * _RPW

    pltpu.sync_copy(lens_hbm, lens_v)
    pltpu.sync_copy(table_hbm.at[pl.ds(0, 1)], t0_v)
    t0 = [t0_v[0, pl.ds(k * 16, 16)] for k in range(_NK)]
    iota = jnp.arange(16, dtype=jnp.int32)
    zf = jnp.zeros((16,), jnp.float32)
    osems = (osem0, osem1)
    bufs = ((a_v, gsema), (b_v, gsemb), (c_v, gsemc))

    def codes_desc():
        return pltpu.make_async_copy(
            codes_hbm.at[pl.ds(0, _C), pl.ds(0, _S), pl.ds(b0, _RPW)],
            ctw_v,
            csem,
        )

    def transpose_row(r, p):
        for h in range((_S + 15) // 16):
            sv = jnp.minimum(h * 16 + iota, _S - 1)
            pos = (h * 16 + iota) * _C
            for c in range(_C):
                code = plsc.load_gather(
                    ctw_v, [jnp.full((16,), c, jnp.int32), sv,
                            jnp.full((16,), r, jnp.int32)])
                plsc.store_scatter(idx_v.at[p], [pos + c], code)

    def out_desc(r, p):
        return pltpu.make_async_copy(
            out_v.at[p],
            out_hbm.at[b0 + r],
            osems[p],
        )

    def gather_descs(t, p, slot):
        ref, sem = bufs[t]
        s0 = _S0[t] * _C
        return [
            pltpu.make_async_copy(
                table_hbm.at[idx_v.at[p].at[pl.ds(s0 + off, n)]],
                ref.at[slot].at[pl.ds(off, n)],
                sem,
            )
            for off, n in _splits(_SZ[t] * _C)
        ]

    def fire_g(t, p, slot):
        for d in gather_descs(t, p, slot):
            d.start()

    def drain_g(t, p, slot):
        for d in gather_descs(t, p, slot):
            d.wait()

    def counts(t, p, lnv):
        s0, sz = _S0[t], _SZ[t]
        for g in range((sz + 15) // 16):
            lvis = g * 16 + iota
            valid = (s0 + lvis) < lnv
            cnt = jnp.zeros((16,), jnp.int32)
            for c in range(_C):
                code = plsc.load_gather(idx_v.at[p],
                                        [(s0 + lvis) * _C + c])
                cnt = cnt + (code > 0).astype(jnp.int32)
            cntf = cnt.astype(jnp.float32)
            recip = jnp.where(valid & (cnt > 0), 1.0 / cntf, 0.0)
            n0 = (_C - cnt).astype(jnp.float32)
            recip_v[pl.ds(g * 16, 16)] = recip
            n0_v[pl.ds(g * 16, 16)] = n0

    def visits(t, p, slot):
        ref, _ = bufs[t]
        s0, sz = _S0[t], _SZ[t]

        def one(v, c2):
            accs = [jnp.zeros((16,), jnp.float32) for _ in range(_NK)]
            r0 = v * _C
            for c in range(_C):
                for k in range(_NK):
                    accs[k] = accs[k] + ref[slot, r0 + c, pl.ds(k * 16, 16)]
            vv = jnp.full((16,), v, jnp.int32)
            rec = plsc.load_gather(recip_v, [vv])
            n0 = plsc.load_gather(n0_v, [vv])
            for k in range(_NK):
                out_v[p, s0 + v, pl.ds(k * 16, 16)] = \
                    (accs[k] - n0 * t0[k]) * rec
            return c2

        lax.fori_loop(0, sz, one, 0)

    def zero_fill(t, p):
        s0, sz = _S0[t], _SZ[t]
        for v in range(sz):
            for k in range(_NK):
                out_v[p, s0 + v, pl.ds(k * 16, 16)] = zf

    def row_body(r, p):
        b = b0 + r

        @pl.when(r >= 2)
        def _():
            out_desc(r - 2, p).wait()

        lnv = plsc.load_gather(lens_v, [jnp.full((16,), b, jnp.int32)])
        ln = jnp.max(lnv)
        live1 = ln > _S0[1]
        live2 = ln > _S0[2]

        @pl.when(live1)
        def _():
            fire_g(1, p, 0)

        counts(0, p, lnv)
        drain_g(0, p, p)

        @pl.when(r + 1 < _RPW)
        def _():
            transpose_row(r + 1, 1 - p)
            fire_g(0, 1 - p, 1 - p)

        visits(0, p, p)

        @pl.when(live2)
        def _():
            fire_g(2, p, 0)

        @pl.when(live1)
        def _():
            counts(1, p, lnv)
            drain_g(1, p, 0)
            visits(1, p, 0)

        @pl.when(jnp.logical_not(live1))
        def _():
            zero_fill(1, p)

        @pl.when(live2)
        def _():
            counts(2, p, lnv)
            drain_g(2, p, 0)
            visits(2, p, 0)

        @pl.when(jnp.logical_not(live2))
        def _():
            zero_fill(2, p)

        out_desc(r, p).start()

    codes_desc().start()
    codes_desc().wait()
    transpose_row(0, 0)
    fire_g(0, 0, 0)

    def pair(i, carry):
        row_body(2 * i, 0)
        row_body(2 * i + 1, 1)
        return carry

    lax.fori_loop(0, _RPW // 2, pair, 0)
    out_desc(_RPW - 2, 0).wait()
    out_desc(_RPW - 1, 1).wait()


_sc_call = functools.partial(
    pl.kernel,
    out_type=jax.ShapeDtypeStruct((_B, _S, _D), jnp.float32),
    mesh=plsc.VectorSubcoreMesh(core_axis_name="c", subcore_axis_name="s"),
    scratch_types=[
        pltpu.VMEM((2, _IDXPAD), jnp.int32),
        pltpu.VMEM((_C, _S, _RPW), jnp.int32),
        pltpu.VMEM((2, _SZ[0] * _C, _D), jnp.float32),
        pltpu.VMEM((1, _SZ[1] * _C, _D), jnp.float32),
        pltpu.VMEM((1, _SZ[2] * _C, _D), jnp.float32),
        pltpu.VMEM((2, _S, _D), jnp.float32),
        pltpu.VMEM((_B,), jnp.int32),
        pltpu.VMEM((32,), jnp.float32),
        pltpu.VMEM((32,), jnp.float32),
        pltpu.VMEM((1, _D), jnp.float32),
        pltpu.SemaphoreType.DMA,
        pltpu.SemaphoreType.DMA,
        pltpu.SemaphoreType.DMA,
        pltpu.SemaphoreType.DMA,
        pltpu.SemaphoreType.DMA,
        pltpu.SemaphoreType.DMA,
    ],
    compiler_params=pltpu.CompilerParams(
        use_tc_tiling_on_sc=False,
        needs_layout_passes=False,
        disable_bounds_checks=True,
        disable_semaphore_checks=True,
    ),
)(_sc_body)


@jax.jit
def kernel(code_embeddings, visit_codes, visit_lens):
    codes_t = jnp.transpose(visit_codes, (2, 1, 0))
    return _sc_call(codes_t, visit_lens, code_embeddings)


# speedup vs baseline: 1.0118x; 1.0118x over previous
import functools

import jax
import jax.numpy as jnp
from jax import lax
from jax.experimental import pallas as pl
from jax.experimental.pallas import tpu as pltpu
from jax.experimental.pallas import tpu_sc as plsc

_S = 50
_C = 20
_D = 64
_B = 1024
_NW = 32
_NV = _B * _S
_RPW = _B // _NW
_SUB = 128
_S0 = (0, 16, 32)
_SZ = (16, 16, 18)
_NK = 4
_IDXPAD = 1280


def _splits(n_rows):
    out, off = [], 0
    while off < n_rows:
        n = min(_SUB, n_rows - off)
        out.append((off, n))
        off += n
    return out


def _sc_body(codes_hbm, lens_hbm, table_hbm, out_hbm,
             idx_v, a_v, b_v, c_v, out_v, lens_v, recip_v, n0_v, t0_v,
             gsema, gsemb, gsemc, csem0, csem1, osem0, osem1):
    nc = 2
    wid = lax.axis_index("s") * nc + lax.axis_index("c")
    b0 = wid * _RPW

    pltpu.sync_copy(lens_hbm, lens_v)
    pltpu.sync_copy(table_hbm.at[pl.ds(0, 1)], t0_v)
    t0 = [t0_v[0, pl.ds(k * 16, 16)] for k in range(_NK)]
    iota = jnp.arange(16, dtype=jnp.int32)
    zf = jnp.zeros((16,), jnp.float32)
    csems = (csem0, csem1)
    osems = (osem0, osem1)
    bufs = ((a_v, gsema), (b_v, gsemb), (c_v, gsemc))

    def codes_desc(r, p):
        return pltpu.make_async_copy(
            codes_hbm.at[pl.ds((b0 + r) * _S * _C, _S * _C)],
            idx_v.at[p].at[pl.ds(0, _S * _C)],
            csems[p],
        )

    def out_desc(r, p):
        return pltpu.make_async_copy(
            out_v.at[p],
            out_hbm.at[b0 + r],
            osems[p],
        )

    def gather_descs(t, p, slot):
        ref, sem = bufs[t]
        s0 = _S0[t] * _C
        return [
            pltpu.make_async_copy(
                table_hbm.at[idx_v.at[p].at[pl.ds(s0 + off, n)]],
                ref.at[slot].at[pl.ds(off, n)],
                sem,
            )
            for off, n in _splits(_SZ[t] * _C)
        ]

    def fire_g(t, p, slot):
        for d in gather_descs(t, p, slot):
            d.start()

    def drain_g(t, p, slot):
        for d in gather_descs(t, p, slot):
            d.wait()

    def counts(t, p, lnv):
        s0, sz = _S0[t], _SZ[t]
        for g in range((sz + 15) // 16):
            lvis = g * 16 + iota
            valid = (s0 + lvis) < lnv
            cnt = jnp.zeros((16,), jnp.int32)
            for c in range(_C):
                code = plsc.load_gather(idx_v.at[p],
                                        [(s0 + lvis) * _C + c])
                cnt = cnt + (code > 0).astype(jnp.int32)
            cntf = cnt.astype(jnp.float32)
            recip = jnp.where(valid & (cnt > 0), 1.0 / cntf, 0.0)
            n0 = (_C - cnt).astype(jnp.float32)
            recip_v[pl.ds(g * 16, 16)] = recip
            n0_v[pl.ds(g * 16, 16)] = n0

    def visits(t, p, slot):
        ref, _ = bufs[t]
        s0, sz = _S0[t], _SZ[t]

        def one(v, c2):
            accs = [jnp.zeros((16,), jnp.float32) for _ in range(_NK)]
            r0 = v * _C
            for c in range(_C):
                for k in range(_NK):
                    accs[k] = accs[k] + ref[slot, r0 + c, pl.ds(k * 16, 16)]
            vv = jnp.full((16,), v, jnp.int32)
            rec = plsc.load_gather(recip_v, [vv])
            n0 = plsc.load_gather(n0_v, [vv])
            for k in range(_NK):
                out_v[p, s0 + v, pl.ds(k * 16, 16)] = \
                    (accs[k] - n0 * t0[k]) * rec
            return c2

        lax.fori_loop(0, sz, one, 0)

    def zero_fill(t, p):
        s0, sz = _S0[t], _SZ[t]
        for v in range(sz):
            for k in range(_NK):
                out_v[p, s0 + v, pl.ds(k * 16, 16)] = zf

    def row_body(r, p):
        b = b0 + r

        @pl.when(r >= 2)
        def _():
            out_desc(r - 2, p).wait()

        lnv = plsc.load_gather(lens_v, [jnp.full((16,), b, jnp.int32)])
        ln = jnp.max(lnv)
        live1 = ln > _S0[1]
        live2 = ln > _S0[2]
        lnv_n = plsc.load_gather(
            lens_v, [jnp.full((16,), jnp.minimum(b + 1, _B - 1), jnp.int32)])
        ln_n = jnp.max(jnp.where(jnp.full((16,), r + 1 < _RPW), lnv_n, 0))
        next1 = ln_n > _S0[1]
        next2 = ln_n > _S0[2]

        counts(0, p, lnv)
        drain_g(0, p, p)

        @pl.when(r + 1 < _RPW)
        def _():
            codes_desc(r + 1, 1 - p).wait()
            fire_g(0, 1 - p, 1 - p)

        visits(0, p, p)

        @pl.when(live1)
        def _():
            counts(1, p, lnv)
            drain_g(1, p, 0)
            visits(1, p, 0)

        @pl.when(jnp.logical_not(live1))
        def _():
            zero_fill(1, p)

        @pl.when(next1)
        def _():
            fire_g(1, 1 - p, 0)

        @pl.when(live2)
        def _():
            counts(2, p, lnv)
            drain_g(2, p, 0)
            visits(2, p, 0)

        @pl.when(jnp.logical_not(live2))
        def _():
            zero_fill(2, p)

        @pl.when(next2)
        def _():
            fire_g(2, 1 - p, 0)

        out_desc(r, p).start()

        @pl.when(r + 2 < _RPW)
        def _():
            codes_desc(r + 2, p).start()

    codes_desc(0, 0).start()
    codes_desc(0, 0).wait()
    fire_g(0, 0, 0)
    lnv0 = plsc.load_gather(lens_v, [jnp.full((16,), b0, jnp.int32)])
    ln0 = jnp.max(lnv0)

    @pl.when(ln0 > _S0[1])
    def _():
        fire_g(1, 0, 0)

    @pl.when(ln0 > _S0[2])
    def _():
        fire_g(2, 0, 0)

    codes_desc(1, 1).start()

    def pair(i, carry):
        row_body(2 * i, 0)
        row_body(2 * i + 1, 1)
        return carry

    lax.fori_loop(0, _RPW // 2, pair, 0)
    out_desc(_RPW - 2, 0).wait()
    out_desc(_RPW - 1, 1).wait()


_sc_call = functools.partial(
    pl.kernel,
    out_type=jax.ShapeDtypeStruct((_B, _S, _D), jnp.float32),
    mesh=plsc.VectorSubcoreMesh(core_axis_name="c", subcore_axis_name="s"),
    scratch_types=[
        pltpu.VMEM((2, _IDXPAD), jnp.int32),
        pltpu.VMEM((2, _SZ[0] * _C, _D), jnp.float32),
        pltpu.VMEM((1, _SZ[1] * _C, _D), jnp.float32),
        pltpu.VMEM((1, _SZ[2] * _C, _D), jnp.float32),
        pltpu.VMEM((2, _S, _D), jnp.float32),
        pltpu.VMEM((_B,), jnp.int32),
        pltpu.VMEM((32,), jnp.float32),
        pltpu.VMEM((32,), jnp.float32),
        pltpu.VMEM((1, _D), jnp.float32),
        pltpu.SemaphoreType.DMA,
        pltpu.SemaphoreType.DMA,
        pltpu.SemaphoreType.DMA,
        pltpu.SemaphoreType.DMA,
        pltpu.SemaphoreType.DMA,
        pltpu.SemaphoreType.DMA,
        pltpu.SemaphoreType.DMA,
    ],
    compiler_params=pltpu.CompilerParams(
        use_tc_tiling_on_sc=False,
        needs_layout_passes=False,
        disable_bounds_checks=True,
        disable_semaphore_checks=True,
    ),
)(_sc_body)


@jax.jit
def kernel(code_embeddings, visit_codes, visit_lens):
    codes_flat = visit_codes.reshape(-1)
    return _sc_call(codes_flat, visit_lens, code_embeddings)


# speedup vs baseline: 1.0371x; 1.0250x over previous
import functools

import jax
import jax.numpy as jnp
from jax import lax
from jax.experimental import pallas as pl
from jax.experimental.pallas import tpu as pltpu
from jax.experimental.pallas import tpu_sc as plsc

_S = 50
_C = 20
_D = 64
_B = 1024
_NW = 32
_NV = _B * _S
_RPW = _B // _NW
_SUB = 128
_S0 = (0, 16, 32)
_SZ = (16, 16, 18)
_NK = 4
_IDXPAD = 1280


def _splits(n_rows):
    out, off = [], 0
    while off < n_rows:
        n = min(_SUB, n_rows - off)
        out.append((off, n))
        off += n
    return out


def _sc_body(codes_hbm, lens_hbm, table_hbm, out_hbm,
             idx_v, a_v, b_v, c_v, out_v, lens_v, recip_v, n0_v, t0_v,
             gsema, gsemb, gsemc, csem0, csem1, osem0, osem1):
    nc = 2
    wid = lax.axis_index("s") * nc + lax.axis_index("c")
    b0 = wid * _RPW

    pltpu.sync_copy(lens_hbm, lens_v)
    pltpu.sync_copy(table_hbm.at[pl.ds(0, 1)], t0_v)
    t0 = [t0_v[0, pl.ds(k * 16, 16)] for k in range(_NK)]
    iota = jnp.arange(16, dtype=jnp.int32)
    zf = jnp.zeros((16,), jnp.float32)
    csems = (csem0, csem1)
    osems = (osem0, osem1)
    bufs = ((a_v, gsema), (b_v, gsemb), (c_v, gsemc))

    def codes_desc(r, p):
        return pltpu.make_async_copy(
            codes_hbm.at[pl.ds((b0 + r) * _S * _C, _S * _C)],
            idx_v.at[p].at[pl.ds(0, _S * _C)],
            csems[p],
        )

    def out_desc(r, p):
        return pltpu.make_async_copy(
            out_v.at[p],
            out_hbm.at[b0 + r],
            osems[p],
        )

    def gather_descs(t, p, slot):
        ref, sem = bufs[t]
        s0 = _S0[t] * _C
        return [
            pltpu.make_async_copy(
                table_hbm.at[idx_v.at[p].at[pl.ds(s0 + off, n)]],
                ref.at[slot].at[pl.ds(off, n)],
                sem,
            )
            for off, n in _splits(_SZ[t] * _C)
        ]

    def fire_g(t, p, slot):
        for d in gather_descs(t, p, slot):
            d.start()

    def drain_g(t, p, slot):
        for d in gather_descs(t, p, slot):
            d.wait()

    def counts(t, p, lnv):
        s0, sz = _S0[t], _SZ[t]
        for g in range((sz + 15) // 16):
            lvis = g * 16 + iota
            valid = (s0 + lvis) < lnv
            cnt = jnp.zeros((16,), jnp.int32)
            for c in range(_C):
                code = plsc.load_gather(idx_v.at[p],
                                        [(s0 + lvis) * _C + c])
                cnt = cnt + (code > 0).astype(jnp.int32)
            cntf = cnt.astype(jnp.float32)
            recip = jnp.where(valid & (cnt > 0), 1.0 / cntf, 0.0)
            n0 = (_C - cnt).astype(jnp.float32)
            recip_v[pl.ds(g * 16, 16)] = recip
            n0_v[pl.ds(g * 16, 16)] = n0

    def visits(t, p, slot):
        ref, _ = bufs[t]
        s0, sz = _S0[t], _SZ[t]

        def one(v, c2):
            accs = [jnp.zeros((16,), jnp.float32) for _ in range(_NK)]
            r0 = v * _C
            for c in range(_C):
                for k in range(_NK):
                    accs[k] = accs[k] + ref[slot, r0 + c, pl.ds(k * 16, 16)]
            vv = jnp.full((16,), v, jnp.int32)
            rec = plsc.load_gather(recip_v, [vv])
            n0 = plsc.load_gather(n0_v, [vv])
            for k in range(_NK):
                out_v[p, s0 + v, pl.ds(k * 16, 16)] = \
                    (accs[k] - n0 * t0[k]) * rec
            return c2

        lax.fori_loop(0, sz, one, 0)

    def zero_fill(t, p):
        s0, sz = _S0[t], _SZ[t]
        for v in range(sz):
            for k in range(_NK):
                out_v[p, s0 + v, pl.ds(k * 16, 16)] = zf

    def row_body(r, p):
        b = b0 + r

        @pl.when(r >= 2)
        def _():
            out_desc(r - 2, p).wait()

        lnv = plsc.load_gather(lens_v, [jnp.full((16,), b, jnp.int32)])
        ln = jnp.max(lnv)
        live1 = ln > _S0[1]
        live2 = ln > _S0[2]

        @pl.when(live1)
        def _():
            fire_g(1, p, 0)

        counts(0, p, lnv)
        drain_g(0, p, p)

        @pl.when(r + 1 < _RPW)
        def _():
            codes_desc(r + 1, 1 - p).wait()
            fire_g(0, 1 - p, 1 - p)

        visits(0, p, p)

        @pl.when(live2)
        def _():
            fire_g(2, p, 0)

        @pl.when(live1)
        def _():
            counts(1, p, lnv)
            drain_g(1, p, 0)
            visits(1, p, 0)

        @pl.when(jnp.logical_not(live1))
        def _():
            zero_fill(1, p)

        @pl.when(live2)
        def _():
            counts(2, p, lnv)
            drain_g(2, p, 0)
            visits(2, p, 0)

        @pl.when(jnp.logical_not(live2))
        def _():
            zero_fill(2, p)

        out_desc(r, p).start()

        @pl.when(r + 2 < _RPW)
        def _():
            codes_desc(r + 2, p).start()

    codes_desc(0, 0).start()
    codes_desc(0, 0).wait()
    fire_g(0, 0, 0)
    codes_desc(1, 1).start()

    def pair(i, carry):
        row_body(2 * i, 0)
        row_body(2 * i + 1, 1)
        return carry

    lax.fori_loop(0, _RPW // 2, pair, 0)
    out_desc(_RPW - 2, 0).wait()
    out_desc(_RPW - 1, 1).wait()


_sc_call = functools.partial(
    pl.kernel,
    out_type=jax.ShapeDtypeStruct((_B, _S, _D), jnp.float32),
    mesh=plsc.VectorSubcoreMesh(core_axis_name="c", subcore_axis_name="s"),
    scratch_types=[
        pltpu.VMEM((2, _IDXPAD), jnp.int32),
        pltpu.VMEM((2, _SZ[0] * _C, _D), jnp.float32),
        pltpu.VMEM((1, _SZ[1] * _C, _D), jnp.float32),
        pltpu.VMEM((1, _SZ[2] * _C, _D), jnp.float32),
        pltpu.VMEM((2, _S, _D), jnp.float32),
        pltpu.VMEM((_B,), jnp.int32),
        pltpu.VMEM((32,), jnp.float32),
        pltpu.VMEM((32,), jnp.float32),
        pltpu.VMEM((1, _D), jnp.float32),
        pltpu.SemaphoreType.DMA,
        pltpu.SemaphoreType.DMA,
        pltpu.SemaphoreType.DMA,
        pltpu.SemaphoreType.DMA,
        pltpu.SemaphoreType.DMA,
        pltpu.SemaphoreType.DMA,
        pltpu.SemaphoreType.DMA,
    ],
    compiler_params=pltpu.CompilerParams(
        use_tc_tiling_on_sc=False,
        needs_layout_passes=False,
        disable_bounds_checks=True,
        disable_semaphore_checks=True,
    ),
)(_sc_body)


@jax.jit
def kernel(code_embeddings, visit_codes, visit_lens):
    codes_flat = visit_codes.reshape(-1)
    return _sc_call(codes_flat, visit_lens, code_embeddings)
